# baseline (speedup 1.0000x reference)
import functools

import jax
import jax.numpy as jnp
from jax import lax
from jax.experimental import pallas as pl
from jax.experimental.pallas import tpu as pltpu
from jax.experimental.pallas import tpu_sc as plsc

VOCAB = 1000000
EMBED = 64
BATCH = 16384
PAIR = 2 * EMBED
NPAIR = VOCAB // 2

NUM_CORES = 2
NUM_SUBCORES = 16
LANES = 16
NUM_WORKERS = NUM_CORES * NUM_SUBCORES

BLK = 128
FULL_BLOCKS = VOCAB // BLK
BASE_BLOCKS = FULL_BLOCKS // NUM_WORKERS
EXTRA_BLOCKS = FULL_BLOCKS % NUM_WORKERS
TAIL_COLS = VOCAB - FULL_BLOCKS * BLK
TAIL_PAIRS = TAIL_COLS // 2

ROWS_PER_WORKER = BATCH // NUM_WORKERS
HALF = ROWS_PER_WORKER // 2
CHUNK = 128
GROUPS = HALF // LANES

_MESH = dict(core_axis_name="c", subcore_axis_name="s",
             num_cores=NUM_CORES, num_subcores=NUM_SUBCORES)
_PARAMS = pltpu.CompilerParams(needs_layout_passes=False,
                               use_tc_tiling_on_sc=True)


def _fmt_body(tt_hbm, out_hbm, inbuf, obuf, tailbuf, sem_in, sem_out):
    wid = lax.axis_index("s") * NUM_CORES + lax.axis_index("c")
    extra = jnp.minimum(wid, EXTRA_BLOCKS)
    start = wid * BASE_BLOCKS + extra
    nblk = jnp.where(wid < EXTRA_BLOCKS, BASE_BLOCKS + 1, BASE_BLOCKS)

    lanes = lax.iota(jnp.int32, LANES)

    def fire_in(s):
        pltpu.async_copy(tt_hbm.at[:, pl.ds((start + s) * BLK, BLK)],
                         inbuf.at[s & 1], sem_in)

    def transpose(b):
        bvec = jnp.full((LANES,), b, jnp.int32)
        for p in range(BLK // 2):
            for j16 in range(PAIR // LANES):
                j0 = j16 * LANES
                e0 = j0 % EMBED
                c = 2 * p + j0 // EMBED
                col = plsc.load_gather(
                    inbuf, [bvec, e0 + lanes, jnp.full((LANES,), c, jnp.int32)])
                obuf[b, p, pl.ds(j0, LANES)] = col

    def drain(sem, ref_slice):
        pltpu.make_async_copy(tt_hbm.at[:, pl.ds(0, BLK)], ref_slice, sem).wait()

    fire_in(jnp.int32(0))

    def step(s, carry):
        @pl.when(s + 1 < nblk)
        def _():
            fire_in(s + 1)
        drain(sem_in, inbuf.at[s & 1])

        @pl.when(s >= 2)
        def _():
            drain(sem_out, obuf.at[s & 1])
        transpose(s & 1)
        pltpu.async_copy(obuf.at[s & 1],
                         out_hbm.at[pl.ds((start + s) * (BLK // 2), BLK // 2)],
                         sem_out)
        return carry

    lax.fori_loop(0, nblk, step, 0)
    drain(sem_out, obuf.at[0])
    drain(sem_out, obuf.at[1])

    @pl.when(wid == NUM_WORKERS - 1)
    def _():
        pltpu.sync_copy(tt_hbm.at[:, pl.ds(FULL_BLOCKS * BLK, TAIL_COLS)],
                        tailbuf)
        for p in range(TAIL_PAIRS):
            for j16 in range(PAIR // LANES):
                j0 = j16 * LANES
                e0 = j0 % EMBED
                c = 2 * p + j0 // EMBED
                col = plsc.load_gather(
                    tailbuf, [e0 + lanes, jnp.full((LANES,), c, jnp.int32)])
                obuf[0, p, pl.ds(j0, LANES)] = col
        pltpu.sync_copy(obuf.at[0, pl.ds(0, TAIL_PAIRS)],
                        out_hbm.at[pl.ds(FULL_BLOCKS * (BLK // 2), TAIL_PAIRS)])


def _dot_body(x_hbm, t_hbm, tbl_hbm, out_hbm,
              idx_x, idx_t, pidx_x, pidx_t, rows_x, rows_t, out_v, sem):
    wid = lax.axis_index("s") * NUM_CORES + lax.axis_index("c")
    base = wid * ROWS_PER_WORKER

    pltpu.sync_copy(x_hbm.at[pl.ds(base, ROWS_PER_WORKER)], idx_x)
    pltpu.sync_copy(t_hbm.at[pl.ds(base, ROWS_PER_WORKER)], idx_t)

    def shift(i, carry):
        pidx_x[pl.ds(i * LANES, LANES)] = idx_x[pl.ds(i * LANES, LANES)] >> 1
        pidx_t[pl.ds(i * LANES, LANES)] = idx_t[pl.ds(i * LANES, LANES)] >> 1
        return carry

    lax.fori_loop(0, ROWS_PER_WORKER // LANES, shift, 0)

    lanes = lax.iota(jnp.int32, LANES)

    def half(h, carry):
        hb = h * HALF
        copies = []
        for j in range(HALF // CHUNK):
            copies.append(pltpu.async_copy(
                tbl_hbm.at[pidx_x.at[pl.ds(hb + j * CHUNK, CHUNK)]],
                rows_x.at[pl.ds(j * CHUNK, CHUNK)], sem))
            copies.append(pltpu.async_copy(
                tbl_hbm.at[pidx_t.at[pl.ds(hb + j * CHUNK, CHUNK)]],
                rows_t.at[pl.ds(j * CHUNK, CHUNK)], sem))
        for c in copies:
            c.wait()

        def group(g, carry2):
            r = hb + g * LANES
            vx = idx_x[pl.ds(r, LANES)]
            vt = idx_t[pl.ds(r, LANES)]
            ridx = g * LANES + lanes
            cx = (vx & 1) * EMBED
            ct = (vt & 1) * EMBED
            acc = jnp.zeros((LANES,), jnp.float32)
            for d in range(EMBED):
                gx = plsc.load_gather(rows_x, [ridx, cx + d])
                gt = plsc.load_gather(rows_t, [ridx, ct + d])
                acc = acc + gx * gt
            out_v[pl.ds(r, LANES)] = acc
            return carry2

        lax.fori_loop(0, GROUPS, group, 0)
        return carry

    lax.fori_loop(0, 2, half, 0)

    pltpu.sync_copy(out_v, out_hbm.at[pl.ds(base, ROWS_PER_WORKER)])


@jax.jit
def kernel(x, t, table):
    mesh = plsc.VectorSubcoreMesh(**_MESH)
    fmt = pl.kernel(
        _fmt_body,
        out_type=jax.ShapeDtypeStruct((NPAIR, PAIR), jnp.float32),
        mesh=mesh,
        scratch_types=[
            pltpu.VMEM((2, EMBED, BLK), jnp.float32),
            pltpu.VMEM((2, BLK // 2, PAIR), jnp.float32),
            pltpu.VMEM((EMBED, TAIL_COLS), jnp.float32),
            pltpu.SemaphoreType.DMA,
            pltpu.SemaphoreType.DMA,
        ],
        compiler_params=_PARAMS,
    )
    dot = pl.kernel(
        _dot_body,
        out_type=jax.ShapeDtypeStruct((BATCH,), jnp.float32),
        mesh=plsc.VectorSubcoreMesh(**_MESH),
        scratch_types=[
            pltpu.VMEM((ROWS_PER_WORKER,), jnp.int32),
            pltpu.VMEM((ROWS_PER_WORKER,), jnp.int32),
            pltpu.VMEM((ROWS_PER_WORKER,), jnp.int32),
            pltpu.VMEM((ROWS_PER_WORKER,), jnp.int32),
            pltpu.VMEM((HALF, PAIR), jnp.float32),
            pltpu.VMEM((HALF, PAIR), jnp.float32),
            pltpu.VMEM((ROWS_PER_WORKER,), jnp.float32),
            pltpu.SemaphoreType.DMA,
        ],
        compiler_params=_PARAMS,
    )
    packed = fmt(table.T)
    return dot(x, t, packed)


# speedup vs baseline: 2.1448x; 2.1448x over previous
import functools

import jax
import jax.numpy as jnp
from jax import lax
from jax.experimental import pallas as pl
from jax.experimental.pallas import tpu as pltpu
from jax.experimental.pallas import tpu_sc as plsc

VOCAB = 1000000
EMBED = 64
BATCH = 16384
PAIR = 2 * EMBED
NPAIR = VOCAB // 2

NUM_CORES = 2
NUM_SUBCORES = 16
LANES = 16
NUM_WORKERS = NUM_CORES * NUM_SUBCORES

BLK = 512
NBLK = (VOCAB + 2 * BLK - 1) // (2 * BLK)
OUT_ROWS = NBLK * BLK

ROWS_PER_WORKER = BATCH // NUM_WORKERS
HALF = ROWS_PER_WORKER // 2
CHUNK = 128
GROUPS = HALF // LANES

_MESH = dict(core_axis_name="c", subcore_axis_name="s",
             num_cores=NUM_CORES, num_subcores=NUM_SUBCORES)
_PARAMS = pltpu.CompilerParams(needs_layout_passes=False,
                               use_tc_tiling_on_sc=True)


def _fmt_body(lo_ref, hi_ref, out_ref):
    out_ref[...] = jnp.concatenate([lo_ref[...].T, hi_ref[...].T], axis=1)


def _dot_body(x_hbm, t_hbm, tbl_hbm, out_hbm,
              idx_x, idx_t, pidx_x, pidx_t, rows_x, rows_t, out_v, sem):
    wid = lax.axis_index("s") * NUM_CORES + lax.axis_index("c")
    base = wid * ROWS_PER_WORKER

    pltpu.sync_copy(x_hbm.at[pl.ds(base, ROWS_PER_WORKER)], idx_x)
    pltpu.sync_copy(t_hbm.at[pl.ds(base, ROWS_PER_WORKER)], idx_t)

    def shift(i, carry):
        vx = idx_x[pl.ds(i * LANES, LANES)]
        vt = idx_t[pl.ds(i * LANES, LANES)]
        pidx_x[pl.ds(i * LANES, LANES)] = (vx >> 10) * BLK + (vx & (BLK - 1))
        pidx_t[pl.ds(i * LANES, LANES)] = (vt >> 10) * BLK + (vt & (BLK - 1))
        return carry

    lax.fori_loop(0, ROWS_PER_WORKER // LANES, shift, 0)

    lanes = lax.iota(jnp.int32, LANES)

    def half(h, carry):
        hb = h * HALF
        copies = []
        for j in range(HALF // CHUNK):
            copies.append(pltpu.async_copy(
                tbl_hbm.at[pidx_x.at[pl.ds(hb + j * CHUNK, CHUNK)]],
                rows_x.at[pl.ds(j * CHUNK, CHUNK)], sem))
            copies.append(pltpu.async_copy(
                tbl_hbm.at[pidx_t.at[pl.ds(hb + j * CHUNK, CHUNK)]],
                rows_t.at[pl.ds(j * CHUNK, CHUNK)], sem))
        for c in copies:
            c.wait()

        def group(g, carry2):
            r = hb + g * LANES
            vx = idx_x[pl.ds(r, LANES)]
            vt = idx_t[pl.ds(r, LANES)]
            ridx = g * LANES + lanes
            cx = ((vx >> 9) & 1) * EMBED
            ct = ((vt >> 9) & 1) * EMBED
            acc = jnp.zeros((LANES,), jnp.float32)
            for d in range(EMBED):
                gx = plsc.load_gather(rows_x, [ridx, cx + d])
                gt = plsc.load_gather(rows_t, [ridx, ct + d])
                acc = acc + gx * gt
            out_v[pl.ds(r, LANES)] = acc
            return carry2

        lax.fori_loop(0, GROUPS, group, 0)
        return carry

    lax.fori_loop(0, 2, half, 0)

    pltpu.sync_copy(out_v, out_hbm.at[pl.ds(base, ROWS_PER_WORKER)])


@jax.jit
def kernel(x, t, table):
    fmt = pl.pallas_call(
        _fmt_body,
        out_shape=jax.ShapeDtypeStruct((OUT_ROWS, PAIR), jnp.float32),
        grid=(NBLK,),
        in_specs=[pl.BlockSpec((EMBED, BLK), lambda i: (0, 2 * i)),
                  pl.BlockSpec((EMBED, BLK), lambda i: (0, 2 * i + 1))],
        out_specs=pl.BlockSpec((BLK, PAIR), lambda i: (i, 0)),
        compiler_params=pltpu.CompilerParams(
            dimension_semantics=("parallel",)),
    )
    dot = pl.kernel(
        _dot_body,
        out_type=jax.ShapeDtypeStruct((BATCH,), jnp.float32),
        mesh=plsc.VectorSubcoreMesh(**_MESH),
        scratch_types=[
            pltpu.VMEM((ROWS_PER_WORKER,), jnp.int32),
            pltpu.VMEM((ROWS_PER_WORKER,), jnp.int32),
            pltpu.VMEM((ROWS_PER_WORKER,), jnp.int32),
            pltpu.VMEM((ROWS_PER_WORKER,), jnp.int32),
            pltpu.VMEM((HALF, PAIR), jnp.float32),
            pltpu.VMEM((HALF, PAIR), jnp.float32),
            pltpu.VMEM((ROWS_PER_WORKER,), jnp.float32),
            pltpu.SemaphoreType.DMA,
        ],
        compiler_params=_PARAMS,
    )
    tt = table.T
    packed = fmt(tt, tt)
    return dot(x, t, packed)


# speedup vs baseline: 2.4401x; 1.1376x over previous
import functools

import jax
import jax.numpy as jnp
from jax import lax
from jax.experimental import pallas as pl
from jax.experimental.pallas import tpu as pltpu
from jax.experimental.pallas import tpu_sc as plsc

VOCAB = 1000000
EMBED = 64
BATCH = 16384
PAIR = 2 * EMBED
NPAIR = VOCAB // 2

NUM_CORES = 2
NUM_SUBCORES = 16
LANES = 16
NUM_WORKERS = NUM_CORES * NUM_SUBCORES

ROWS_PER_WORKER = BATCH // NUM_WORKERS
HALF = ROWS_PER_WORKER // 2
CHUNK = 128
GROUPS = HALF // LANES

_MESH = dict(core_axis_name="c", subcore_axis_name="s",
             num_cores=NUM_CORES, num_subcores=NUM_SUBCORES)
_PARAMS = pltpu.CompilerParams(needs_layout_passes=False,
                               use_tc_tiling_on_sc=True)


def _dot_body(x_hbm, t_hbm, tbl_hbm, out_hbm,
              idx_x, idx_t, pidx_x, pidx_t, rows_x, rows_t, out_v, sem):
    wid = lax.axis_index("s") * NUM_CORES + lax.axis_index("c")
    base = wid * ROWS_PER_WORKER

    pltpu.sync_copy(x_hbm.at[pl.ds(base, ROWS_PER_WORKER)], idx_x)
    pltpu.sync_copy(t_hbm.at[pl.ds(base, ROWS_PER_WORKER)], idx_t)

    def shift(i, carry):
        pidx_x[pl.ds(i * LANES, LANES)] = idx_x[pl.ds(i * LANES, LANES)] >> 1
        pidx_t[pl.ds(i * LANES, LANES)] = idx_t[pl.ds(i * LANES, LANES)] >> 1
        return carry

    lax.fori_loop(0, ROWS_PER_WORKER // LANES, shift, 0)

    lanes = lax.iota(jnp.int32, LANES)

    def half(h, carry):
        hb = h * HALF
        copies = []
        for j in range(HALF // CHUNK):
            copies.append(pltpu.async_copy(
                tbl_hbm.at[pidx_x.at[pl.ds(hb + j * CHUNK, CHUNK)]],
                rows_x.at[pl.ds(j * CHUNK, CHUNK)], sem))
            copies.append(pltpu.async_copy(
                tbl_hbm.at[pidx_t.at[pl.ds(hb + j * CHUNK, CHUNK)]],
                rows_t.at[pl.ds(j * CHUNK, CHUNK)], sem))
        for c in copies:
            c.wait()

        def group(g, carry2):
            r = hb + g * LANES
            vx = idx_x[pl.ds(r, LANES)]
            vt = idx_t[pl.ds(r, LANES)]
            ridx = g * LANES + lanes
            cx = (vx & 1) * EMBED
            ct = (vt & 1) * EMBED
            acc = jnp.zeros((LANES,), jnp.float32)
            for d in range(EMBED):
                gx = plsc.load_gather(rows_x, [ridx, cx + d])
                gt = plsc.load_gather(rows_t, [ridx, ct + d])
                acc = acc + gx * gt
            out_v[pl.ds(r, LANES)] = acc
            return carry2

        lax.fori_loop(0, GROUPS, group, 0)
        return carry

    lax.fori_loop(0, 2, half, 0)

    pltpu.sync_copy(out_v, out_hbm.at[pl.ds(base, ROWS_PER_WORKER)])


@jax.jit
def kernel(x, t, table):
    dot = pl.kernel(
        _dot_body,
        out_type=jax.ShapeDtypeStruct((BATCH,), jnp.float32),
        mesh=plsc.VectorSubcoreMesh(**_MESH),
        scratch_types=[
            pltpu.VMEM((ROWS_PER_WORKER,), jnp.int32),
            pltpu.VMEM((ROWS_PER_WORKER,), jnp.int32),
            pltpu.VMEM((ROWS_PER_WORKER,), jnp.int32),
            pltpu.VMEM((ROWS_PER_WORKER,), jnp.int32),
            pltpu.VMEM((HALF, PAIR), jnp.float32),
            pltpu.VMEM((HALF, PAIR), jnp.float32),
            pltpu.VMEM((ROWS_PER_WORKER,), jnp.float32),
            pltpu.SemaphoreType.DMA,
        ],
        compiler_params=_PARAMS,
    )
    packed = jnp.reshape(table, (NPAIR, PAIR))
    return dot(x, t, packed)
